# Initial kernel scaffold; baseline (speedup 1.0000x reference)
import jax
import jax.numpy as jnp
from jax.experimental import pallas as pl


def kernel(x, edge_index, epsilon, g1, b1, m1, v1, W1, a_src, a_dst, bias1, g2, b2, m2, v2, W2, bias2, g3, b3, m3, v3, W3, bias3, Wm, bm, Wv, bv):
    raise NotImplementedError("write your pallas kernel here")


# speedup vs baseline: 6.9800x; 6.9800x over previous
import functools

import jax
import jax.numpy as jnp
from jax import lax
from jax.experimental import pallas as pl
from jax.experimental.pallas import tpu as pltpu
from jax.experimental.pallas import tpu_sc as plsc

N_NODES = 10000
N_EDGES = 320000
DIM = 128
HID = 128
H2 = 64
H3 = 32
LATENT = 64

NPAD = 10240
WGAT = 144
NW = 32
NS = 16
EPAD = 327680
EPT = EPAD // NW
CH = 128
NCH = EPT // CH
ROWS_PER_TILE = NPAD // NS
BLK = 128
GRID = NPAD // BLK

_mesh = plsc.VectorSubcoreMesh(core_axis_name="c", subcore_axis_name="s")
_sc_params = pltpu.CompilerParams(use_tc_tiling_on_sc=False,
                                  needs_layout_passes=False)


def _zero_rows(rows_v, width):
    iota = lax.iota(jnp.int32, 16)
    zeros = jnp.zeros((16,), jnp.float32)

    def body(r, _):
        ridx = jnp.full((16,), r, jnp.int32)
        for j in range(width // 16):
            plsc.store_scatter(rows_v, [ridx, iota + (16 * j)], zeros)
        return 0

    lax.fori_loop(0, CH, body, 0)


def _init_acc(rows_v, acc_sh, s, width):
    _zero_rows(rows_v, width)
    for i in range(ROWS_PER_TILE // CH):
        pltpu.sync_copy(rows_v, acc_sh.at[pl.ds(s * ROWS_PER_TILE + i * CH, CH)])


@functools.partial(
    pl.kernel,
    out_type=jax.ShapeDtypeStruct((2, NPAD, WGAT), jnp.float32),
    mesh=_mesh,
    scratch_types=[
        pltpu.VMEM((2, CH), jnp.int32),
        pltpu.VMEM((CH, WGAT), jnp.float32),
        pltpu.VMEM((CH,), jnp.float32),
        pltpu.VMEM((CH,), jnp.float32),
        pltpu.VMEM((CH,), jnp.float32),
        pltpu.VMEM_SHARED((NPAD, WGAT), jnp.float32),
        pltpu.SemaphoreType.DMA,
        pltpu.SemaphoreType.DMA,
    ],
    compiler_params=_sc_params,
)
def _gat_sc(hpad_hbm, src_hbm, dst_hbm, ssrc_hbm, sdst_hbm, out_hbm,
            idx_v, rows_v, ee_v, sbuf_v, dbuf_v, acc_sh, sem, sem2):
    c = lax.axis_index("c")
    s = lax.axis_index("s")
    wid = c * NS + s

    _init_acc(rows_v, acc_sh, s, WGAT)
    plsc.subcore_barrier()

    iota = lax.iota(jnp.int32, 16)

    def chunk(i, _):
        off = wid * EPT + i * CH
        pltpu.sync_copy(src_hbm.at[pl.ds(off, CH)], idx_v.at[0])
        pltpu.sync_copy(dst_hbm.at[pl.ds(off, CH)], idx_v.at[1])
        cp_rows = pltpu.async_copy(hpad_hbm.at[idx_v.at[0]], rows_v, sem)
        cp_s = pltpu.async_copy(ssrc_hbm.at[idx_v.at[0]], sbuf_v, sem2)
        cp_d = pltpu.async_copy(sdst_hbm.at[idx_v.at[1]], dbuf_v, sem2)
        cp_rows.wait()
        cp_s.wait()
        cp_d.wait()
        for g in range(CH // 16):
            ev = sbuf_v[pl.ds(g * 16, 16)] + dbuf_v[pl.ds(g * 16, 16)]
            ev = jnp.maximum(ev, 0.2 * ev)
            ee_v[pl.ds(g * 16, 16)] = jnp.exp(ev)
        def rowscale(r, _):
            ridx = jnp.full((16,), r, jnp.int32)
            w = plsc.load_gather(ee_v, [ridx])
            for j in range(8):
                cols = iota + (16 * j)
                v = plsc.load_gather(rows_v, [ridx, cols])
                plsc.store_scatter(rows_v, [ridx, cols], v * w)
            cols = iota + 128
            m = jnp.where(iota == 0, w, 1.0)
            v = plsc.load_gather(rows_v, [ridx, cols])
            plsc.store_scatter(rows_v, [ridx, cols], v * m)
            return 0

        lax.fori_loop(0, CH, rowscale, 0)
        pltpu.sync_copy(rows_v, acc_sh.at[idx_v.at[1]], add=True)
        return 0

    lax.fori_loop(0, NCH, chunk, 0)
    plsc.subcore_barrier()
    pltpu.sync_copy(acc_sh.at[pl.ds(s * ROWS_PER_TILE, ROWS_PER_TILE)],
                    out_hbm.at[c, pl.ds(s * ROWS_PER_TILE, ROWS_PER_TILE)])


def _make_gcn_sc(width):
    @functools.partial(
        pl.kernel,
        out_type=jax.ShapeDtypeStruct((2, NPAD, width), jnp.float32),
        mesh=_mesh,
        scratch_types=[
            pltpu.VMEM((2, CH), jnp.int32),
            pltpu.VMEM((CH, width), jnp.float32),
            pltpu.VMEM_SHARED((NPAD, width), jnp.float32),
            pltpu.SemaphoreType.DMA,
        ],
        compiler_params=_sc_params,
    )
    def gcn_sc(h_hbm, src_hbm, dst_hbm, out_hbm, idx_v, rows_v, acc_sh, sem):
        c = lax.axis_index("c")
        s = lax.axis_index("s")
        wid = c * NS + s

        _init_acc(rows_v, acc_sh, s, width)
        plsc.subcore_barrier()

        def chunk(i, _):
            off = wid * EPT + i * CH
            pltpu.sync_copy(src_hbm.at[pl.ds(off, CH)], idx_v.at[0])
            pltpu.sync_copy(dst_hbm.at[pl.ds(off, CH)], idx_v.at[1])
            pltpu.async_copy(h_hbm.at[idx_v.at[0]], rows_v, sem).wait()
            pltpu.sync_copy(rows_v, acc_sh.at[idx_v.at[1]], add=True)
            return 0

        lax.fori_loop(0, NCH, chunk, 0)
        plsc.subcore_barrier()
        pltpu.sync_copy(acc_sh.at[pl.ds(s * ROWS_PER_TILE, ROWS_PER_TILE)],
                        out_hbm.at[c, pl.ds(s * ROWS_PER_TILE, ROWS_PER_TILE)])

    return gcn_sc


_gcn_sc64 = _make_gcn_sc(H2)
_gcn_sc32 = _make_gcn_sc(H3)


def _bn_fold(h, g_ref, b_ref, m_ref, v_ref):
    scale = g_ref[...] * lax.rsqrt(v_ref[...] + 1e-3)
    return h * scale + (b_ref[...] - m_ref[...] * scale)


def _tc1_body(x_ref, w1_ref, asrc_ref, adst_ref, g_ref, b_ref, m_ref, v_ref,
              hpad_ref, ssrc_ref, sdst_ref):
    xb = _bn_fold(x_ref[...], g_ref, b_ref, m_ref, v_ref)
    h = jnp.dot(xb, w1_ref[...], preferred_element_type=jnp.float32)
    hpad_ref[:, :HID] = h
    l16 = lax.broadcasted_iota(jnp.int32, (BLK, WGAT - HID), 1)
    hpad_ref[:, HID:WGAT] = jnp.where(l16 < 2, 1.0, 0.0)
    ssrc_ref[...] = jnp.sum(h * asrc_ref[...], axis=1, keepdims=True)
    sdst_ref[...] = jnp.sum(h * adst_ref[...], axis=1, keepdims=True)


def _tc2_body(a0_ref, a1_ref, bias1_ref, g_ref, b_ref, m_ref, v_ref, w2_ref,
              hs2_ref, rdeg_ref):
    a = a0_ref[...] + a1_ref[...]
    cols = a[:, :HID]
    tail = a[:, HID:WGAT]
    l16 = lax.broadcasted_iota(jnp.int32, (BLK, WGAT - HID), 1)
    denom = jnp.sum(jnp.where(l16 == 0, tail, 0.0), axis=1, keepdims=True)
    deg = jnp.sum(jnp.where(l16 == 1, tail, 0.0), axis=1, keepdims=True)
    out1 = jax.nn.relu(cols / (denom + 1e-9) + bias1_ref[...])
    h2 = jnp.dot(_bn_fold(out1, g_ref, b_ref, m_ref, v_ref), w2_ref[...],
                 preferred_element_type=jnp.float32)
    rdeg = lax.rsqrt(jnp.maximum(deg, 1.0))
    hs2_ref[...] = h2 * rdeg
    rdeg_ref[...] = rdeg


def _tc3_body(a0_ref, a1_ref, rdeg_ref, bias2_ref, g_ref, b_ref, m_ref, v_ref,
              w3_ref, hs3_ref):
    rdeg = rdeg_ref[...]
    out2 = jax.nn.relu(rdeg * (a0_ref[...] + a1_ref[...]) + bias2_ref[...])
    h3 = jnp.dot(_bn_fold(out2, g_ref, b_ref, m_ref, v_ref), w3_ref[...],
                 preferred_element_type=jnp.float32)
    hs3_ref[...] = h3 * rdeg


def _tc4_body(a0_ref, a1_ref, rdeg_ref, eps_ref, bias3_ref, wm_ref, bm_ref,
              wv_ref, bv_ref, zm_ref, zlv_ref, z_ref):
    out3 = jax.nn.relu(rdeg_ref[...] * (a0_ref[...] + a1_ref[...])
                       + bias3_ref[...])
    zm = jax.nn.sigmoid(jnp.dot(out3, wm_ref[...],
                                preferred_element_type=jnp.float32)
                        + bm_ref[...])
    zlv = jnp.dot(out3, wv_ref[...], preferred_element_type=jnp.float32) \
        + bv_ref[...]
    zm_ref[...] = zm
    zlv_ref[...] = zlv
    z_ref[...] = zm + jnp.exp(0.5 * zlv) * eps_ref[...]


def _row_spec(width):
    return pl.BlockSpec((BLK, width), lambda i: (i, 0))


def _full_spec(shape):
    nd = len(shape)
    return pl.BlockSpec(shape, lambda i: (0,) * nd)


def _vec_spec(width):
    return _full_spec((1, width))


def kernel(x, edge_index, epsilon, g1, b1, m1, v1, W1, a_src, a_dst, bias1,
           g2, b2, m2, v2, W2, bias2, g3, b3, m3, v3, W3, bias3,
           Wm, bm, Wv, bv):
    f32 = jnp.float32
    src = edge_index[0].astype(jnp.int32)
    dst = edge_index[1].astype(jnp.int32)
    srcp = jnp.concatenate([src, jnp.zeros((EPAD - N_EDGES,), jnp.int32)])
    dstp = jnp.concatenate([dst, jnp.full((EPAD - N_EDGES,), N_NODES,
                                          jnp.int32)])
    xp = jnp.pad(x, ((0, NPAD - N_NODES), (0, 0)))
    epsp = jnp.pad(epsilon, ((0, NPAD - N_NODES), (0, 0)))

    def row(v):
        return v.reshape(1, -1).astype(f32)

    tc1 = pl.pallas_call(
        _tc1_body,
        grid=(GRID,),
        in_specs=[_row_spec(DIM), _full_spec((DIM, HID)), _vec_spec(HID),
                  _vec_spec(HID), _vec_spec(DIM), _vec_spec(DIM),
                  _vec_spec(DIM), _vec_spec(DIM)],
        out_specs=[_row_spec(WGAT), _row_spec(1), _row_spec(1)],
        out_shape=[jax.ShapeDtypeStruct((NPAD, WGAT), f32),
                   jax.ShapeDtypeStruct((NPAD, 1), f32),
                   jax.ShapeDtypeStruct((NPAD, 1), f32)],
    )
    hpad, ssrc, sdst = tc1(xp, W1, row(a_src), row(a_dst), row(g1), row(b1),
                           row(m1), row(v1))

    accg = _gat_sc(hpad, srcp, dstp, ssrc.reshape(NPAD), sdst.reshape(NPAD))

    tc2 = pl.pallas_call(
        _tc2_body,
        grid=(GRID,),
        in_specs=[_row_spec(WGAT), _row_spec(WGAT), _vec_spec(HID),
                  _vec_spec(HID), _vec_spec(HID), _vec_spec(HID),
                  _vec_spec(HID), _full_spec((HID, H2))],
        out_specs=[_row_spec(H2), _row_spec(1)],
        out_shape=[jax.ShapeDtypeStruct((NPAD, H2), f32),
                   jax.ShapeDtypeStruct((NPAD, 1), f32)],
    )
    hs2, rdeg1 = tc2(accg[0], accg[1], row(bias1), row(g2), row(b2), row(m2),
                     row(v2), W2)

    acc2 = _gcn_sc64(hs2, srcp, dstp)

    tc3 = pl.pallas_call(
        _tc3_body,
        grid=(GRID,),
        in_specs=[_row_spec(H2), _row_spec(H2), _row_spec(1), _vec_spec(H2),
                  _vec_spec(H2), _vec_spec(H2), _vec_spec(H2), _vec_spec(H2),
                  _full_spec((H2, H3))],
        out_specs=[_row_spec(H3)],
        out_shape=[jax.ShapeDtypeStruct((NPAD, H3), f32)],
    )
    hs3, = tc3(acc2[0], acc2[1], rdeg1, row(bias2), row(g3), row(b3), row(m3),
               row(v3), W3)

    acc3 = _gcn_sc32(hs3, srcp, dstp)

    tc4 = pl.pallas_call(
        _tc4_body,
        grid=(GRID,),
        in_specs=[_row_spec(H3), _row_spec(H3), _row_spec(1),
                  _row_spec(LATENT), _vec_spec(H3), _full_spec((H3, LATENT)),
                  _vec_spec(LATENT), _full_spec((H3, LATENT)),
                  _vec_spec(LATENT)],
        out_specs=[_row_spec(LATENT), _row_spec(LATENT), _row_spec(LATENT)],
        out_shape=[jax.ShapeDtypeStruct((NPAD, LATENT), f32),
                   jax.ShapeDtypeStruct((NPAD, LATENT), f32),
                   jax.ShapeDtypeStruct((NPAD, LATENT), f32)],
    )
    zm, zlv, z = tc4(acc3[0], acc3[1], rdeg1, epsp, row(bias3), Wm, row(bm),
                     Wv, row(bv))

    return zm[:N_NODES], zlv[:N_NODES], z[:N_NODES]
